# baseline (speedup 1.0000x reference)
import functools

import jax
import jax.numpy as jnp
from jax import lax
from jax.experimental import pallas as pl
from jax.experimental.pallas import tpu as pltpu
from jax.experimental.pallas import tpu_sc as plsc

N_, C_, H_, W_ = 16, 4, 512, 512
LN = 128
GROWS = (H_ * W_) // LN
NPIX = N_ * H_ * W_
RB = 512
BINS = 16384


def _p1_body(p1, p2, p3, t_ref, mu1_o, mu2_o, mu3_o, lo1_o, lo2_o, lo3_o,
             acc_o):
    i = pl.program_id(0)
    j = pl.program_id(1)
    t = t_ref[0]

    def ce(pref):
        l0 = pref[0, 0]
        l1 = pref[0, 1]
        l2 = pref[0, 2]
        l3 = pref[0, 3]
        m = jnp.maximum(jnp.maximum(l0, l1), jnp.maximum(l2, l3))
        s = (jnp.exp(l0 - m) + jnp.exp(l1 - m)
             + jnp.exp(l2 - m) + jnp.exp(l3 - m))
        lse = m + jnp.log(s)
        lt = jnp.where(t == 0, l0,
                       jnp.where(t == 1, l1, jnp.where(t == 2, l2, l3)))
        return jnp.where(t == -1, 0.0, lse - lt)

    loss1 = ce(p1)
    loss2 = ce(p2)
    loss3 = ce(p3)
    fg = (t == 2) | (t == 3)
    bg = (t == 0) | (t == 1)
    inf = jnp.float32(jnp.inf)

    def binify(mu):
        return lax.shift_right_logical(
            lax.bitcast_convert_type(mu, jnp.int32), 17).astype(jnp.int16)

    mu1_o[0] = binify(jnp.where(fg, loss1 + jnp.abs(loss2 - loss3), inf))
    mu2_o[0] = binify(jnp.where(fg, loss2 + jnp.abs(loss3 - loss1), inf))
    mu3_o[0] = binify(jnp.where(fg, loss3 + jnp.abs(loss1 - loss2), inf))
    lo1_o[0] = loss1.astype(jnp.bfloat16)
    lo2_o[0] = loss2.astype(jnp.bfloat16)
    lo3_o[0] = loss3.astype(jnp.bfloat16)

    @pl.when((i == 0) & (j == 0))
    def _():
        acc_o[...] = jnp.zeros_like(acc_o)

    zero = jnp.float32(0.0)
    acc_o[0] += jnp.sum(jnp.where(bg, loss1, zero), axis=0, keepdims=True)
    acc_o[1] += jnp.sum(jnp.where(bg, loss2, zero), axis=0, keepdims=True)
    acc_o[2] += jnp.sum(jnp.where(bg, loss3, zero), axis=0, keepdims=True)
    acc_o[3] += jnp.sum(fg.astype(jnp.float32), axis=0, keepdims=True)


def _pass1(p1, p2, p3, t):
    pix = jax.ShapeDtypeStruct((N_, GROWS, LN), jnp.bfloat16)
    pixi = jax.ShapeDtypeStruct((N_, GROWS, LN), jnp.int16)
    grid = (N_, GROWS // RB)
    pspec = pl.BlockSpec((1, C_, RB, LN), lambda i, j: (i, 0, j, 0))
    tspec = pl.BlockSpec((1, RB, LN), lambda i, j: (i, j, 0))
    ospec = pl.BlockSpec((1, RB, LN), lambda i, j: (i, j, 0))
    aspec = pl.BlockSpec((4, 1, LN), lambda i, j: (0, 0, 0))
    return pl.pallas_call(
        _p1_body,
        grid=grid,
        in_specs=[pspec, pspec, pspec, tspec],
        out_specs=[ospec, ospec, ospec, ospec, ospec, ospec, aspec],
        out_shape=[pixi, pixi, pixi, pix, pix, pix,
                   jax.ShapeDtypeStruct((4, 1, LN), jnp.float32)],
    )(p1, p2, p3, t)


NCORE = 2
NSUB = 16
NTILE = NCORE * NSUB
NWORD = NPIX // 2
PER_TILE = NWORD // NTILE
CHUNK = 4096


UNROLL = 8


def _hist_body(mu1, mu2, mu3, lo1, lo2, lo3, cnt_out, ls_out,
               mub0, mub1, lob0, lob1, c1, c2, c3, s1, s2, s3,
               msem0, msem1, lsem0, lsem1):
    cid = lax.axis_index("c")
    sid = lax.axis_index("s")
    wid = sid * NCORE + cid
    base = wid * PER_TILE
    mubufs = (mub0, mub1)
    lobufs = (lob0, lob1)
    msems = (msem0, msem1)
    lsems = (lsem0, lsem1)
    NCH = PER_TILE // CHUNK

    zi = jnp.zeros((16,), jnp.int32)
    zf = jnp.zeros((16,), jnp.float32)

    def zero_body(i, _):
        idx = pl.ds(i * 16, 16)
        c1[idx] = zi
        c2[idx] = zi
        c3[idx] = zi
        s1[idx] = zf
        s2[idx] = zf
        s3[idx] = zf
        return 0

    lax.fori_loop(0, BINS // 16, zero_body, 0)

    ones = jnp.full((16,), 1, dtype=jnp.int32)

    for mu_hbm, lo_hbm, ch, sh in ((mu1, lo1, c1, s1),
                                   (mu2, lo2, c2, s2),
                                   (mu3, lo3, c3, s3)):
        def start(c, b, mu_hbm=mu_hbm, lo_hbm=lo_hbm):
            off = base + c * CHUNK
            pltpu.async_copy(mu_hbm.at[pl.ds(off, CHUNK)], mubufs[b],
                             msems[b])
            pltpu.async_copy(lo_hbm.at[pl.ds(off, CHUNK)], lobufs[b],
                             lsems[b])

        def wait(c, b, mu_hbm=mu_hbm, lo_hbm=lo_hbm):
            off = base + c * CHUNK
            pltpu.make_async_copy(mu_hbm.at[pl.ds(off, CHUNK)], mubufs[b],
                                  msems[b]).wait()
            pltpu.make_async_copy(lo_hbm.at[pl.ds(off, CHUNK)], lobufs[b],
                                  lsems[b]).wait()

        def compute(b, ch=ch, sh=sh):
            mub = mubufs[b]
            lob = lobufs[b]

            def grp_body(g, _):
                bs = []
                lv = []
                mask16 = jnp.full((16,), 0xFFFF, dtype=jnp.int32)
                hi16 = jnp.full((16,), -65536, dtype=jnp.int32)
                for u in range(UNROLL):
                    idx = pl.ds(g * (16 * UNROLL) + u * 16, 16)
                    bw = mub[idx]
                    lw = lob[idx]
                    bs.append((bw & mask16,
                               lax.shift_right_logical(bw, 16)))
                    lf_lo = plsc.bitcast(
                        lax.shift_left(lw, 16), jnp.float32)
                    lf_hi = plsc.bitcast(lw & hi16, jnp.float32)
                    lv.append((lf_lo, lf_hi))
                for u in range(UNROLL):
                    plsc.addupdate_scatter(ch, [bs[u][0]], ones)
                    plsc.addupdate_scatter(sh, [bs[u][0]], lv[u][0])
                    plsc.addupdate_scatter(ch, [bs[u][1]], ones)
                    plsc.addupdate_scatter(sh, [bs[u][1]], lv[u][1])
                return 0

            lax.fori_loop(0, CHUNK // (16 * UNROLL), grp_body, 0)

        start(0, 0)
        start(1, 1)

        def pair_body(i, _):
            c0 = 2 * i
            wait(c0, 0)
            compute(0)

            @pl.when(c0 + 2 < NCH)
            def _():
                start(c0 + 2, 0)

            wait(c0 + 1, 1)
            compute(1)

            @pl.when(c0 + 3 < NCH)
            def _():
                start(c0 + 3, 1)

            return 0

        lax.fori_loop(0, NCH // 2, pair_body, 0)

    for a, (ch, sh) in enumerate(((c1, s1), (c2, s2), (c3, s3))):
        pltpu.sync_copy(ch, cnt_out.at[a, wid])
        pltpu.sync_copy(sh, ls_out.at[a, wid])


def _sc_hists(mu1, mu2, mu3, lo1, lo2, lo3):
    mesh = plsc.VectorSubcoreMesh(core_axis_name="c", subcore_axis_name="s")
    f = pl.kernel(
        _hist_body,
        out_type=[jax.ShapeDtypeStruct((3, NTILE, BINS), jnp.int32),
                  jax.ShapeDtypeStruct((3, NTILE, BINS), jnp.float32)],
        mesh=mesh,
        compiler_params=pltpu.CompilerParams(needs_layout_passes=False),
        scratch_types=[
            pltpu.VMEM((CHUNK,), jnp.int32),
            pltpu.VMEM((CHUNK,), jnp.int32),
            pltpu.VMEM((CHUNK,), jnp.int32),
            pltpu.VMEM((CHUNK,), jnp.int32),
            pltpu.VMEM((BINS,), jnp.int32),
            pltpu.VMEM((BINS,), jnp.int32),
            pltpu.VMEM((BINS,), jnp.int32),
            pltpu.VMEM((BINS,), jnp.float32),
            pltpu.VMEM((BINS,), jnp.float32),
            pltpu.VMEM((BINS,), jnp.float32),
            pltpu.SemaphoreType.DMA,
            pltpu.SemaphoreType.DMA,
            pltpu.SemaphoreType.DMA,
            pltpu.SemaphoreType.DMA,
        ],
    )
    def pack(x):
        return lax.bitcast_convert_type(x.reshape(NWORD, 2), jnp.int32)

    return f(pack(mu1), pack(mu2), pack(mu3),
             pack(lo1), pack(lo2), pack(lo3))


def kernel(preds1, preds2, preds3, target, epoch):
    t = target.astype(jnp.int32).reshape(N_, GROWS, LN)
    p1 = preds1.reshape(N_, C_, GROWS, LN)
    p2 = preds2.reshape(N_, C_, GROWS, LN)
    p3 = preds3.reshape(N_, C_, GROWS, LN)
    mu1, mu2, mu3, lo1, lo2, lo3, acc = _pass1(p1, p2, p3, t)

    n_fg = jnp.sum(acc[3]).astype(jnp.int32)
    num_remember = (n_fg.astype(jnp.float32) * 0.5).astype(jnp.int32)
    num = NPIX - n_fg + num_remember

    cnt_t, ls_t = _sc_hists(mu1, mu2, mu3, lo1, lo2, lo3)
    cnt = jnp.sum(cnt_t, axis=1)
    ls = jnp.sum(ls_t, axis=1)

    def sel_sum(cnt_i, ls_i):
        inc = jnp.cumsum(cnt_i)
        b = jnp.searchsorted(inc, num_remember, side='left')
        cnt_below = inc[b] - cnt_i[b]
        lsum_below = jnp.cumsum(ls_i)[b] - ls_i[b]
        f = (num_remember - cnt_below).astype(jnp.float32) / jnp.maximum(
            cnt_i[b], 1).astype(jnp.float32)
        return lsum_below + f * ls_i[b]

    outs = []
    for idx in range(3):
        bg_sum = jnp.sum(acc[idx])
        outs.append((sel_sum(cnt[idx], ls[idx]) + bg_sum) / num)
    return tuple(outs)


# speedup vs baseline: 11.7755x; 11.7755x over previous
import functools

import jax
import jax.numpy as jnp
from jax import lax
from jax.experimental import pallas as pl
from jax.experimental.pallas import tpu as pltpu
from jax.experimental.pallas import tpu_sc as plsc

N_, C_, H_, W_ = 16, 4, 512, 512
LN = 128
GROWS = (H_ * W_) // LN
NPIX = N_ * H_ * W_
RB = 512
BINS = 16384


def _p1_body(p1, p2, p3, t_ref, mu1_o, mu2_o, mu3_o, lo1_o, lo2_o, lo3_o,
             acc_o):
    i = pl.program_id(0)
    j = pl.program_id(1)
    t = t_ref[0]

    def ce(pref):
        l0 = pref[0, 0]
        l1 = pref[0, 1]
        l2 = pref[0, 2]
        l3 = pref[0, 3]
        m = jnp.maximum(jnp.maximum(l0, l1), jnp.maximum(l2, l3))
        s = (jnp.exp(l0 - m) + jnp.exp(l1 - m)
             + jnp.exp(l2 - m) + jnp.exp(l3 - m))
        lse = m + jnp.log(s)
        lt = jnp.where(t == 0, l0,
                       jnp.where(t == 1, l1, jnp.where(t == 2, l2, l3)))
        return jnp.where(t == -1, 0.0, lse - lt)

    loss1 = ce(p1)
    loss2 = ce(p2)
    loss3 = ce(p3)
    fg = (t == 2) | (t == 3)
    bg = (t == 0) | (t == 1)
    inf = jnp.float32(jnp.inf)

    half = RB // 2
    low16 = jnp.full((half, LN), 0xFFFF, dtype=jnp.int32)

    def pack2(x32):
        return (x32[:half] & low16) | lax.shift_left(x32[half:], 16)

    def binify(mu):
        return pack2(lax.shift_right_logical(
            lax.bitcast_convert_type(mu, jnp.int32), 17))

    def packloss(loss):
        b16 = lax.bitcast_convert_type(
            loss.astype(jnp.bfloat16), jnp.int16).astype(jnp.int32)
        return pack2(b16)

    mu1_o[0] = binify(jnp.where(fg, loss1 + jnp.abs(loss2 - loss3), inf))
    mu2_o[0] = binify(jnp.where(fg, loss2 + jnp.abs(loss3 - loss1), inf))
    mu3_o[0] = binify(jnp.where(fg, loss3 + jnp.abs(loss1 - loss2), inf))
    lo1_o[0] = packloss(loss1)
    lo2_o[0] = packloss(loss2)
    lo3_o[0] = packloss(loss3)

    @pl.when((i == 0) & (j == 0))
    def _():
        acc_o[...] = jnp.zeros_like(acc_o)

    zero = jnp.float32(0.0)
    acc_o[0] += jnp.sum(jnp.where(bg, loss1, zero), axis=0, keepdims=True)
    acc_o[1] += jnp.sum(jnp.where(bg, loss2, zero), axis=0, keepdims=True)
    acc_o[2] += jnp.sum(jnp.where(bg, loss3, zero), axis=0, keepdims=True)
    acc_o[3] += jnp.sum(fg.astype(jnp.float32), axis=0, keepdims=True)


def _pass1(p1, p2, p3, t):
    pixp = jax.ShapeDtypeStruct((N_, GROWS // 2, LN), jnp.int32)
    grid = (N_, GROWS // RB)
    pspec = pl.BlockSpec((1, C_, RB, LN), lambda i, j: (i, 0, j, 0))
    tspec = pl.BlockSpec((1, RB, LN), lambda i, j: (i, j, 0))
    ospec = pl.BlockSpec((1, RB // 2, LN), lambda i, j: (i, j, 0))
    aspec = pl.BlockSpec((4, 1, LN), lambda i, j: (0, 0, 0))
    return pl.pallas_call(
        _p1_body,
        grid=grid,
        in_specs=[pspec, pspec, pspec, tspec],
        out_specs=[ospec, ospec, ospec, ospec, ospec, ospec, aspec],
        out_shape=[pixp, pixp, pixp, pixp, pixp, pixp,
                   jax.ShapeDtypeStruct((4, 1, LN), jnp.float32)],
    )(p1, p2, p3, t)


NCORE = 2
NSUB = 16
NTILE = NCORE * NSUB
NWORD = NPIX // 2
PER_TILE = NWORD // NTILE
CHUNK = 4096


UNROLL = 8


def _hist_body(mu1, mu2, mu3, lo1, lo2, lo3, cnt_out, ls_out,
               mub0, mub1, lob0, lob1, c1, c2, c3, s1, s2, s3,
               msem0, msem1, lsem0, lsem1):
    cid = lax.axis_index("c")
    sid = lax.axis_index("s")
    wid = sid * NCORE + cid
    base = wid * PER_TILE
    mubufs = (mub0, mub1)
    lobufs = (lob0, lob1)
    msems = (msem0, msem1)
    lsems = (lsem0, lsem1)
    NCH = PER_TILE // CHUNK

    zi = jnp.zeros((16,), jnp.int32)
    zf = jnp.zeros((16,), jnp.float32)

    def zero_body(i, _):
        idx = pl.ds(i * 16, 16)
        c1[idx] = zi
        c2[idx] = zi
        c3[idx] = zi
        s1[idx] = zf
        s2[idx] = zf
        s3[idx] = zf
        return 0

    lax.fori_loop(0, BINS // 16, zero_body, 0)

    ones = jnp.full((16,), 1, dtype=jnp.int32)

    for mu_hbm, lo_hbm, ch, sh in ((mu1, lo1, c1, s1),
                                   (mu2, lo2, c2, s2),
                                   (mu3, lo3, c3, s3)):
        def start(c, b, mu_hbm=mu_hbm, lo_hbm=lo_hbm):
            off = base + c * CHUNK
            pltpu.async_copy(mu_hbm.at[pl.ds(off, CHUNK)], mubufs[b],
                             msems[b])
            pltpu.async_copy(lo_hbm.at[pl.ds(off, CHUNK)], lobufs[b],
                             lsems[b])

        def wait(c, b, mu_hbm=mu_hbm, lo_hbm=lo_hbm):
            off = base + c * CHUNK
            pltpu.make_async_copy(mu_hbm.at[pl.ds(off, CHUNK)], mubufs[b],
                                  msems[b]).wait()
            pltpu.make_async_copy(lo_hbm.at[pl.ds(off, CHUNK)], lobufs[b],
                                  lsems[b]).wait()

        def compute(b, ch=ch, sh=sh):
            mub = mubufs[b]
            lob = lobufs[b]

            def grp_body(g, _):
                bs = []
                lv = []
                mask16 = jnp.full((16,), 0xFFFF, dtype=jnp.int32)
                hi16 = jnp.full((16,), -65536, dtype=jnp.int32)
                for u in range(UNROLL):
                    idx = pl.ds(g * (16 * UNROLL) + u * 16, 16)
                    bw = mub[idx]
                    lw = lob[idx]
                    bs.append((bw & mask16,
                               lax.shift_right_logical(bw, 16)))
                    lf_lo = plsc.bitcast(
                        lax.shift_left(lw, 16), jnp.float32)
                    lf_hi = plsc.bitcast(lw & hi16, jnp.float32)
                    lv.append((lf_lo, lf_hi))
                for u in range(UNROLL):
                    plsc.addupdate_scatter(ch, [bs[u][0]], ones)
                    plsc.addupdate_scatter(sh, [bs[u][0]], lv[u][0])
                    plsc.addupdate_scatter(ch, [bs[u][1]], ones)
                    plsc.addupdate_scatter(sh, [bs[u][1]], lv[u][1])
                return 0

            lax.fori_loop(0, CHUNK // (16 * UNROLL), grp_body, 0)

        start(0, 0)
        start(1, 1)

        def pair_body(i, _):
            c0 = 2 * i
            wait(c0, 0)
            compute(0)

            @pl.when(c0 + 2 < NCH)
            def _():
                start(c0 + 2, 0)

            wait(c0 + 1, 1)
            compute(1)

            @pl.when(c0 + 3 < NCH)
            def _():
                start(c0 + 3, 1)

            return 0

        lax.fori_loop(0, NCH // 2, pair_body, 0)

    for a, (ch, sh) in enumerate(((c1, s1), (c2, s2), (c3, s3))):
        pltpu.sync_copy(ch, cnt_out.at[a, wid])
        pltpu.sync_copy(sh, ls_out.at[a, wid])


def _sc_hists(mu1, mu2, mu3, lo1, lo2, lo3):
    mesh = plsc.VectorSubcoreMesh(core_axis_name="c", subcore_axis_name="s")
    f = pl.kernel(
        _hist_body,
        out_type=[jax.ShapeDtypeStruct((3, NTILE, BINS), jnp.int32),
                  jax.ShapeDtypeStruct((3, NTILE, BINS), jnp.float32)],
        mesh=mesh,
        compiler_params=pltpu.CompilerParams(needs_layout_passes=False),
        scratch_types=[
            pltpu.VMEM((CHUNK,), jnp.int32),
            pltpu.VMEM((CHUNK,), jnp.int32),
            pltpu.VMEM((CHUNK,), jnp.int32),
            pltpu.VMEM((CHUNK,), jnp.int32),
            pltpu.VMEM((BINS,), jnp.int32),
            pltpu.VMEM((BINS,), jnp.int32),
            pltpu.VMEM((BINS,), jnp.int32),
            pltpu.VMEM((BINS,), jnp.float32),
            pltpu.VMEM((BINS,), jnp.float32),
            pltpu.VMEM((BINS,), jnp.float32),
            pltpu.SemaphoreType.DMA,
            pltpu.SemaphoreType.DMA,
            pltpu.SemaphoreType.DMA,
            pltpu.SemaphoreType.DMA,
        ],
    )
    return f(mu1.reshape(NWORD), mu2.reshape(NWORD), mu3.reshape(NWORD),
             lo1.reshape(NWORD), lo2.reshape(NWORD), lo3.reshape(NWORD))


def kernel(preds1, preds2, preds3, target, epoch):
    t = target.astype(jnp.int32).reshape(N_, GROWS, LN)
    p1 = preds1.reshape(N_, C_, GROWS, LN)
    p2 = preds2.reshape(N_, C_, GROWS, LN)
    p3 = preds3.reshape(N_, C_, GROWS, LN)
    mu1, mu2, mu3, lo1, lo2, lo3, acc = _pass1(p1, p2, p3, t)

    n_fg = jnp.sum(acc[3]).astype(jnp.int32)
    num_remember = (n_fg.astype(jnp.float32) * 0.5).astype(jnp.int32)
    num = NPIX - n_fg + num_remember

    cnt_t, ls_t = _sc_hists(mu1, mu2, mu3, lo1, lo2, lo3)
    cnt = jnp.sum(cnt_t, axis=1)
    ls = jnp.sum(ls_t, axis=1)

    def sel_sum(cnt_i, ls_i):
        inc = jnp.cumsum(cnt_i)
        b = jnp.searchsorted(inc, num_remember, side='left')
        cnt_below = inc[b] - cnt_i[b]
        lsum_below = jnp.cumsum(ls_i)[b] - ls_i[b]
        f = (num_remember - cnt_below).astype(jnp.float32) / jnp.maximum(
            cnt_i[b], 1).astype(jnp.float32)
        return lsum_below + f * ls_i[b]

    outs = []
    for idx in range(3):
        bg_sum = jnp.sum(acc[idx])
        outs.append((sel_sum(cnt[idx], ls[idx]) + bg_sum) / num)
    return tuple(outs)
